# baseline (speedup 1.0000x reference)
import dataclasses
import functools

import jax
import jax.numpy as jnp
import numpy as np
from jax import lax
from jax.experimental import pallas as pl
from jax.experimental.pallas import tpu as pltpu
from jax.experimental.pallas import tpu_sc as plsc

N = 10000
ACC = 10240
IN_CH = 128
HID = 256
OUT_CH = 128
E = 320000
NC = 2
NS = 16
NW = NC * NS
W = 128
NWIN = 80
HN = NWIN // 2
RW = E // W
PADW = NW * NWIN - RW
BM = 5000

_PAD_SRC = np.arange(PADW * W, dtype=np.int32).reshape(PADW, W) % N
_PAD_DST = (N + np.arange(PADW * W, dtype=np.int32).reshape(PADW, W) % (ACC - N)).astype(np.int32)


def _mesh():
    return plsc.VectorSubcoreMesh(core_axis_name="c", subcore_axis_name="s")


def _sc_params():
    cp = pltpu.CompilerParams()
    if "needs_layout_passes" in pltpu.CompilerParams.__dataclass_fields__:
        cp = dataclasses.replace(cp, needs_layout_passes=False)
    return cp


def _sc_degree(ei_win):
    @functools.partial(
        pl.kernel,
        mesh=_mesh(),
        compiler_params=_sc_params(),
        out_type=jax.ShapeDtypeStruct((NC, ACC), jnp.float32),
        scratch_types=[
            pltpu.VMEM((NWIN, W), jnp.int32),
            pltpu.VMEM((ACC,), jnp.float32),
            pltpu.VMEM((640,), jnp.float32),
            pltpu.VMEM((640,), jnp.float32),
            pltpu.VMEM_SHARED((NS, ACC), jnp.float32),
            pltpu.SemaphoreType.DMA,
        ],
    )
    def k(ei_hbm, out_hbm, di_v, hist, tbuf, sbuf, stage, sem):
        cid = lax.axis_index("c")
        sid = lax.axis_index("s")
        wid = sid * NC + cid
        pltpu.sync_copy(ei_hbm.at[1, pl.ds(wid * NWIN, NWIN)], di_v)

        @pl.loop(0, ACC, step=16)
        def _(i):
            hist[pl.ds(i, 16)] = jnp.zeros((16,), jnp.float32)

        ones = jnp.ones((16,), jnp.float32)

        @pl.loop(0, NWIN)
        def _(j):
            @pl.loop(0, W, step=16)
            def _(c0):
                idx = di_v[j, pl.ds(c0, 16)]
                plsc.addupdate_scatter(hist, [idx], ones)

        pltpu.sync_copy(hist, stage.at[sid])
        plsc.subcore_barrier()

        base = sid * 640

        @pl.loop(0, 640, step=16)
        def _(i):
            sbuf[pl.ds(i, 16)] = jnp.zeros((16,), jnp.float32)

        @pl.loop(0, NS)
        def _(t):
            pltpu.sync_copy(stage.at[t, pl.ds(base, 640)], tbuf)

            @pl.loop(0, 640, step=16)
            def _(i):
                sbuf[pl.ds(i, 16)] = sbuf[pl.ds(i, 16)] + tbuf[pl.ds(i, 16)]

        pltpu.sync_copy(sbuf, out_hbm.at[cid, pl.ds(base, 640)])

    return k(ei_win)


_SC_SCATTER_CACHE = []


def _sc_scatter(values, ei_win):
    if _SC_SCATTER_CACHE:
        return _SC_SCATTER_CACHE[0](values, ei_win)

    @functools.partial(
        pl.kernel,
        mesh=_mesh(),
        compiler_params=_sc_params(),
        out_type=jax.ShapeDtypeStruct((NC, ACC, IN_CH), jnp.float32),
        scratch_types=[
            pltpu.VMEM((HN, W), jnp.int32),
            pltpu.VMEM((HN, W), jnp.int32),
            pltpu.VMEM((W, IN_CH), jnp.float32),
            pltpu.VMEM((W, IN_CH), jnp.float32),
            pltpu.SemaphoreType.DMA,
            pltpu.SemaphoreType.DMA,
            pltpu.VMEM_SHARED((ACC, IN_CH), jnp.float32),
        ],
    )
    def k(v_hbm, ei_hbm, out_hbm, si_v, di_v, buf0, buf1, sg0, sg1, acc):
        cid = lax.axis_index("c")
        sid = lax.axis_index("s")
        wid = sid * NC + cid

        @pl.loop(0, W)
        def _(r):
            @pl.loop(0, IN_CH, step=16)
            def _(c0):
                buf0[r, pl.ds(c0, 16)] = jnp.zeros((16,), jnp.float32)

        base = sid * 640

        @pl.loop(0, 5)
        def _(t):
            pltpu.sync_copy(buf0, acc.at[pl.ds(base + t * W, W)])

        plsc.subcore_barrier()

        for h in (0, 1):
            g0 = wid * NWIN + h * HN
            pltpu.sync_copy(ei_hbm.at[0, pl.ds(g0, HN)], si_v)
            pltpu.sync_copy(ei_hbm.at[1, pl.ds(g0, HN)], di_v)

            pltpu.async_copy(v_hbm.at[si_v.at[0]], buf0, sg0)
            pltpu.async_copy(v_hbm.at[si_v.at[1]], buf1, sg1)

            @pl.loop(0, HN - 2, step=2)
            def _(w):
                pltpu.make_async_copy(v_hbm.at[si_v.at[w]], buf0, sg0).wait()
                pltpu.sync_copy(buf0, acc.at[di_v.at[w]], add=True)
                pltpu.async_copy(v_hbm.at[si_v.at[w + 2]], buf0, sg0)
                pltpu.make_async_copy(v_hbm.at[si_v.at[w + 1]], buf1, sg1).wait()
                pltpu.sync_copy(buf1, acc.at[di_v.at[w + 1]], add=True)
                pltpu.async_copy(v_hbm.at[si_v.at[w + 3]], buf1, sg1)

            pltpu.make_async_copy(v_hbm.at[si_v.at[HN - 2]], buf0, sg0).wait()
            pltpu.sync_copy(buf0, acc.at[di_v.at[HN - 2]], add=True)
            pltpu.make_async_copy(v_hbm.at[si_v.at[HN - 1]], buf1, sg1).wait()
            pltpu.sync_copy(buf1, acc.at[di_v.at[HN - 1]], add=True)

        plsc.subcore_barrier()
        pltpu.sync_copy(acc.at[pl.ds(base, 640)], out_hbm.at[cid, pl.ds(base, 640)])

    _SC_SCATTER_CACHE.append(k)
    return k(values, ei_win)


def _tc_scale(x, deg_parts):
    def body(x_ref, d_ref, p_ref, dv_ref):
        deg = d_ref[0] + d_ref[1] + 1.0
        dinv = lax.rsqrt(deg)
        dv_ref[...] = dinv
        p_ref[...] = x_ref[...] * dinv

    return pl.pallas_call(
        body,
        grid=(N // BM,),
        in_specs=[
            pl.BlockSpec((BM, IN_CH), lambda i: (i, 0)),
            pl.BlockSpec((NC, BM, 1), lambda i: (0, i, 0)),
        ],
        out_specs=[
            pl.BlockSpec((BM, IN_CH), lambda i: (i, 0)),
            pl.BlockSpec((BM, 1), lambda i: (i, 0)),
        ],
        out_shape=[
            jax.ShapeDtypeStruct((N, IN_CH), jnp.float32),
            jax.ShapeDtypeStruct((N, 1), jnp.float32),
        ],
    )(x, deg_parts)


def _tc_mid(s1, p, dinv, W1, b1, W2):
    def body(s_ref, p_ref, d_ref, w1_ref, b1_ref, w2_ref, g_ref):
        a = (s_ref[0] + s_ref[1] + p_ref[...]) * d_ref[...]
        h = jnp.dot(a, w1_ref[...], preferred_element_type=jnp.float32)
        h = jnp.maximum(h + b1_ref[...], 0.0)
        g = jnp.dot(h, w2_ref[...], preferred_element_type=jnp.float32)
        g_ref[...] = g * d_ref[...]

    return pl.pallas_call(
        body,
        grid=(N // BM,),
        in_specs=[
            pl.BlockSpec((NC, BM, IN_CH), lambda i: (0, i, 0)),
            pl.BlockSpec((BM, IN_CH), lambda i: (i, 0)),
            pl.BlockSpec((BM, 1), lambda i: (i, 0)),
            pl.BlockSpec((IN_CH, HID), lambda i: (0, 0)),
            pl.BlockSpec((1, HID), lambda i: (0, 0)),
            pl.BlockSpec((HID, OUT_CH), lambda i: (0, 0)),
        ],
        out_specs=pl.BlockSpec((BM, OUT_CH), lambda i: (i, 0)),
        out_shape=jax.ShapeDtypeStruct((N, OUT_CH), jnp.float32),
    )(s1, p, dinv, W1, b1, W2)


def _tc_out(s2, g, dinv, b2):
    def body(s_ref, g_ref, d_ref, b2_ref, o_ref):
        o_ref[...] = (s_ref[0] + s_ref[1] + g_ref[...]) * d_ref[...] + b2_ref[...]

    return pl.pallas_call(
        body,
        grid=(N // BM,),
        in_specs=[
            pl.BlockSpec((NC, BM, OUT_CH), lambda i: (0, i, 0)),
            pl.BlockSpec((BM, OUT_CH), lambda i: (i, 0)),
            pl.BlockSpec((BM, 1), lambda i: (i, 0)),
            pl.BlockSpec((1, OUT_CH), lambda i: (0, 0)),
        ],
        out_specs=pl.BlockSpec((BM, OUT_CH), lambda i: (i, 0)),
        out_shape=jax.ShapeDtypeStruct((N, OUT_CH), jnp.float32),
    )(s2, g, dinv, b2)


def kernel(x, edge_index, W1, b1, W2, b2):
    ei_win = jnp.concatenate(
        [edge_index.astype(jnp.int32).reshape(2, RW, W),
         np.stack([_PAD_SRC, _PAD_DST])], axis=1)

    deg_parts = _sc_degree(ei_win)
    p, dinv = _tc_scale(x, deg_parts[:, :N, None])
    s1 = _sc_scatter(p, ei_win)
    g = _tc_mid(s1, p, dinv, W1, b1.reshape(1, HID), W2)
    s2 = _sc_scatter(g, ei_win)
    return _tc_out(s2, g, dinv, b2.reshape(1, OUT_CH))


# speedup vs baseline: 1.0060x; 1.0060x over previous
import dataclasses
import functools

import jax
import jax.numpy as jnp
import numpy as np
from jax import lax
from jax.experimental import pallas as pl
from jax.experimental.pallas import tpu as pltpu
from jax.experimental.pallas import tpu_sc as plsc

N = 10000
ACC = 10240
IN_CH = 128
HID = 256
OUT_CH = 128
E = 320000
NC = 2
NS = 16
NW = NC * NS
W = 128
NWIN = 80
HN = NWIN // 2
RW = E // W
PADW = NW * NWIN - RW
BM = 5000

_PAD_SRC = np.arange(PADW * W, dtype=np.int32).reshape(PADW, W) % N
_PAD_DST = (N + np.arange(PADW * W, dtype=np.int32).reshape(PADW, W) % (ACC - N)).astype(np.int32)


def _mesh():
    return plsc.VectorSubcoreMesh(core_axis_name="c", subcore_axis_name="s")


def _sc_params():
    cp = pltpu.CompilerParams()
    if "needs_layout_passes" in pltpu.CompilerParams.__dataclass_fields__:
        cp = dataclasses.replace(cp, needs_layout_passes=False)
    return cp


def _sc_degree(ei_win):
    @functools.partial(
        pl.kernel,
        mesh=_mesh(),
        compiler_params=_sc_params(),
        out_type=jax.ShapeDtypeStruct((NC, ACC // 16, 16), jnp.float32),
        scratch_types=[
            pltpu.VMEM((NWIN, W), jnp.int32),
            pltpu.VMEM((ACC // 16, 16), jnp.float32),
            pltpu.VMEM((5, W), jnp.int32),
            pltpu.VMEM((40, 16), jnp.float32),
            pltpu.VMEM_SHARED((ACC // 16, 16), jnp.float32),
        ],
    )
    def k(ei_hbm, out_hbm, di_v, hist, idx, zbuf, acc2, ):
        cid = lax.axis_index("c")
        sid = lax.axis_index("s")
        wid = sid * NC + cid
        pltpu.sync_copy(ei_hbm.at[1, pl.ds(wid * NWIN, NWIN)], di_v)

        lane = jnp.arange(16, dtype=jnp.int32)

        @pl.loop(0, ACC // 16)
        def _(r):
            hist[r, pl.ds(0, 16)] = jnp.zeros((16,), jnp.float32)

        @pl.loop(0, 5)
        def _(r):
            @pl.loop(0, W, step=16)
            def _(c0):
                idx[r, pl.ds(c0, 16)] = lane + r * W + c0

        @pl.loop(0, 40)
        def _(r):
            zbuf[r, pl.ds(0, 16)] = jnp.zeros((16,), jnp.float32)

        pltpu.sync_copy(zbuf, acc2.at[pl.ds(sid * 40, 40)])
        plsc.subcore_barrier()

        ones = jnp.ones((16,), jnp.float32)

        @pl.loop(0, NWIN)
        def _(j):
            @pl.loop(0, W, step=16)
            def _(c0):
                d = di_v[j, pl.ds(c0, 16)]
                plsc.addupdate_scatter(
                    hist,
                    [lax.shift_right_logical(d, 4), lax.bitwise_and(d, 15)],
                    ones,
                )

        @pl.loop(0, 5)
        def _(w):
            pltpu.sync_copy(hist.at[pl.ds(w * W, W)], acc2.at[idx.at[w]], add=True)

        plsc.subcore_barrier()
        pltpu.sync_copy(acc2.at[pl.ds(sid * 40, 40)], out_hbm.at[cid, pl.ds(sid * 40, 40)])

    return k(ei_win)


_SC_SCATTER_CACHE = []


def _sc_scatter(values, ei_win):
    if _SC_SCATTER_CACHE:
        return _SC_SCATTER_CACHE[0](values, ei_win)

    @functools.partial(
        pl.kernel,
        mesh=_mesh(),
        compiler_params=_sc_params(),
        out_type=jax.ShapeDtypeStruct((NC, ACC, IN_CH), jnp.float32),
        scratch_types=[
            pltpu.VMEM((HN, W), jnp.int32),
            pltpu.VMEM((HN, W), jnp.int32),
            pltpu.VMEM((W, IN_CH), jnp.float32),
            pltpu.VMEM((W, IN_CH), jnp.float32),
            pltpu.SemaphoreType.DMA,
            pltpu.SemaphoreType.DMA,
            pltpu.VMEM_SHARED((ACC, IN_CH), jnp.float32),
        ],
    )
    def k(v_hbm, ei_hbm, out_hbm, si_v, di_v, buf0, buf1, sg0, sg1, acc):
        cid = lax.axis_index("c")
        sid = lax.axis_index("s")
        wid = sid * NC + cid

        @pl.loop(0, W)
        def _(r):
            @pl.loop(0, IN_CH, step=16)
            def _(c0):
                buf0[r, pl.ds(c0, 16)] = jnp.zeros((16,), jnp.float32)

        base = sid * 640

        @pl.loop(0, 5)
        def _(t):
            pltpu.sync_copy(buf0, acc.at[pl.ds(base + t * W, W)])

        plsc.subcore_barrier()

        for h in (0, 1):
            g0 = wid * NWIN + h * HN
            pltpu.sync_copy(ei_hbm.at[0, pl.ds(g0, HN)], si_v)
            pltpu.sync_copy(ei_hbm.at[1, pl.ds(g0, HN)], di_v)

            pltpu.async_copy(v_hbm.at[si_v.at[0]], buf0, sg0)
            pltpu.async_copy(v_hbm.at[si_v.at[1]], buf1, sg1)

            @pl.loop(0, HN - 2, step=2)
            def _(w):
                pltpu.make_async_copy(v_hbm.at[si_v.at[w]], buf0, sg0).wait()
                pltpu.sync_copy(buf0, acc.at[di_v.at[w]], add=True)
                pltpu.async_copy(v_hbm.at[si_v.at[w + 2]], buf0, sg0)
                pltpu.make_async_copy(v_hbm.at[si_v.at[w + 1]], buf1, sg1).wait()
                pltpu.sync_copy(buf1, acc.at[di_v.at[w + 1]], add=True)
                pltpu.async_copy(v_hbm.at[si_v.at[w + 3]], buf1, sg1)

            pltpu.make_async_copy(v_hbm.at[si_v.at[HN - 2]], buf0, sg0).wait()
            pltpu.sync_copy(buf0, acc.at[di_v.at[HN - 2]], add=True)
            pltpu.make_async_copy(v_hbm.at[si_v.at[HN - 1]], buf1, sg1).wait()
            pltpu.sync_copy(buf1, acc.at[di_v.at[HN - 1]], add=True)

        plsc.subcore_barrier()
        pltpu.sync_copy(acc.at[pl.ds(base, 640)], out_hbm.at[cid, pl.ds(base, 640)])

    _SC_SCATTER_CACHE.append(k)
    return k(values, ei_win)


def _tc_scale(x, deg_parts):
    def body(x_ref, d_ref, p_ref, dv_ref):
        deg = d_ref[0] + d_ref[1] + 1.0
        dinv = lax.rsqrt(deg)
        dv_ref[...] = dinv
        p_ref[...] = x_ref[...] * dinv

    return pl.pallas_call(
        body,
        grid=(N // BM,),
        in_specs=[
            pl.BlockSpec((BM, IN_CH), lambda i: (i, 0)),
            pl.BlockSpec((NC, BM, 1), lambda i: (0, i, 0)),
        ],
        out_specs=[
            pl.BlockSpec((BM, IN_CH), lambda i: (i, 0)),
            pl.BlockSpec((BM, 1), lambda i: (i, 0)),
        ],
        out_shape=[
            jax.ShapeDtypeStruct((N, IN_CH), jnp.float32),
            jax.ShapeDtypeStruct((N, 1), jnp.float32),
        ],
    )(x, deg_parts)


def _tc_mid(s1, p, dinv, W1, b1, W2):
    def body(s_ref, p_ref, d_ref, w1_ref, b1_ref, w2_ref, g_ref):
        a = (s_ref[0] + s_ref[1] + p_ref[...]) * d_ref[...]
        h = jnp.dot(a, w1_ref[...], preferred_element_type=jnp.float32)
        h = jnp.maximum(h + b1_ref[...], 0.0)
        g = jnp.dot(h, w2_ref[...], preferred_element_type=jnp.float32)
        g_ref[...] = g * d_ref[...]

    return pl.pallas_call(
        body,
        grid=(N // BM,),
        in_specs=[
            pl.BlockSpec((NC, BM, IN_CH), lambda i: (0, i, 0)),
            pl.BlockSpec((BM, IN_CH), lambda i: (i, 0)),
            pl.BlockSpec((BM, 1), lambda i: (i, 0)),
            pl.BlockSpec((IN_CH, HID), lambda i: (0, 0)),
            pl.BlockSpec((1, HID), lambda i: (0, 0)),
            pl.BlockSpec((HID, OUT_CH), lambda i: (0, 0)),
        ],
        out_specs=pl.BlockSpec((BM, OUT_CH), lambda i: (i, 0)),
        out_shape=jax.ShapeDtypeStruct((N, OUT_CH), jnp.float32),
    )(s1, p, dinv, W1, b1, W2)


def _tc_out(s2, g, dinv, b2):
    def body(s_ref, g_ref, d_ref, b2_ref, o_ref):
        o_ref[...] = (s_ref[0] + s_ref[1] + g_ref[...]) * d_ref[...] + b2_ref[...]

    return pl.pallas_call(
        body,
        grid=(N // BM,),
        in_specs=[
            pl.BlockSpec((NC, BM, OUT_CH), lambda i: (0, i, 0)),
            pl.BlockSpec((BM, OUT_CH), lambda i: (i, 0)),
            pl.BlockSpec((BM, 1), lambda i: (i, 0)),
            pl.BlockSpec((1, OUT_CH), lambda i: (0, 0)),
        ],
        out_specs=pl.BlockSpec((BM, OUT_CH), lambda i: (i, 0)),
        out_shape=jax.ShapeDtypeStruct((N, OUT_CH), jnp.float32),
    )(s2, g, dinv, b2)


def kernel(x, edge_index, W1, b1, W2, b2):
    ei_win = jnp.concatenate(
        [edge_index.astype(jnp.int32).reshape(2, RW, W),
         np.stack([_PAD_SRC, _PAD_DST])], axis=1)

    deg_parts = _sc_degree(ei_win).reshape(NC, ACC)
    p, dinv = _tc_scale(x, deg_parts[:, :N, None])
    s1 = _sc_scatter(p, ei_win)
    g = _tc_mid(s1, p, dinv, W1, b1.reshape(1, HID), W2)
    s2 = _sc_scatter(g, ei_win)
    return _tc_out(s2, g, dinv, b2.reshape(1, OUT_CH))
